# baseline (device time: 31257 ns/iter reference)
import jax
import jax.numpy as jnp
from jax import lax
from jax.experimental import pallas as pl
from jax.experimental.pallas import tpu as pltpu

N_DEV = 4
SQ = 256
SKV = 4096
HQ_PER = 8
DH = 128
DM = 1024
HALF = DM // 2
SCALE = 0.08838834764831843


def kernel(x, Wq, K_ext, V_ext, Wo):
    def body(x_ref, wq_ref, k_hbm, v_hbm, wo_ref, out_ref,
             k_vmem, v_vmem, sbuf, rbuf, kv_sems, send_sems, recv_sems):
        my_i = lax.axis_index("i")
        h0 = my_i * HQ_PER
        p1 = my_i ^ 1
        p2 = (N_DEV - 1) - my_i

        k_copies, v_copies = [], []
        for h in range(HQ_PER):
            kc = pltpu.make_async_copy(
                k_hbm.at[0, :, h0 + h, :], k_vmem.at[h], kv_sems.at[0, h])
            vc = pltpu.make_async_copy(
                v_hbm.at[0, :, h0 + h, :], v_vmem.at[h], kv_sems.at[1, h])
            kc.start()
            vc.start()
            k_copies.append(kc)
            v_copies.append(vc)

        xb = x_ref[0].astype(jnp.bfloat16)
        q = jnp.dot(xb, wq_ref[...].astype(jnp.bfloat16),
                    preferred_element_type=jnp.float32)
        q = (q * SCALE).astype(jnp.bfloat16)

        barrier = pltpu.get_barrier_semaphore()
        for p in (p1, p2):
            pl.semaphore_signal(barrier, inc=1, device_id=(p,),
                                device_id_type=pl.DeviceIdType.MESH)
        pl.semaphore_wait(barrier, 2)

        B03 = list(range(0, 64, 3))
        B1 = [0, 1] + list(range(2, 64, 3))
        B2 = [0, 2] + list(range(1, 64, 3))

        def gather(m3, blist):
            return jnp.concatenate([m3[b] for b in blist], axis=0)

        def attend(qg, kg, vg):
            s = lax.dot_general(qg, kg, (((1,), (1,)), ((), ())),
                                preferred_element_type=jnp.float32)
            w = jnp.exp(s)
            wsum = jnp.sum(w, axis=1, keepdims=True)
            cu = jnp.dot(w.astype(jnp.bfloat16), vg,
                         preferred_element_type=jnp.float32)
            return cu / wsum

        acc = jnp.zeros((SQ, DM), jnp.float32)
        for h in range(HQ_PER):
            k_copies[h].wait()
            v_copies[h].wait()
            qh = q[:, h * DH:(h + 1) * DH]
            k3 = k_vmem[h].reshape(64, 64, DH).astype(jnp.bfloat16)
            v3 = v_vmem[h].reshape(64, 64, DH).astype(jnp.bfloat16)
            q03 = jnp.concatenate([qh[0:64], qh[192:256]], axis=0)
            c03 = attend(q03, gather(k3, B03), gather(v3, B03))
            c1 = attend(qh[64:128], gather(k3, B1), gather(v3, B1))
            c2 = attend(qh[128:192], gather(k3, B2), gather(v3, B2))
            ctx_h = jnp.concatenate(
                [c03[0:64], c1, c2, c03[64:128]], axis=0)
            wo_h = wo_ref[h * DH:(h + 1) * DH, :].astype(jnp.bfloat16)
            acc = acc + jnp.dot(ctx_h.astype(jnp.bfloat16), wo_h,
                                preferred_element_type=jnp.float32)

        QC = DM // 4

        def xchg(slot, peer, payload):
            sbuf[slot] = payload.astype(jnp.bfloat16)
            r = pltpu.make_async_remote_copy(
                src_ref=sbuf.at[slot], dst_ref=rbuf.at[slot],
                send_sem=send_sems.at[slot], recv_sem=recv_sems.at[slot],
                device_id=(peer,), device_id_type=pl.DeviceIdType.MESH)
            r.start()
            return r

        quarters = [acc[:, c * QC:(c + 1) * QC] for c in range(4)]
        peers1 = (p1, p1, p2, p2)
        peers2 = (p2, p2, p1, p1)
        ph1 = [xchg(c, peers1[c], quarters[c]) for c in range(4)]

        acc1 = [None] * 4
        ph2 = [None] * 4
        for c in (0, 2, 1, 3):
            ph1[c].wait_recv()
            acc1[c] = quarters[c] + rbuf[c].astype(jnp.float32)
            ph2[c] = xchg(4 + c, peers2[c], acc1[c])
        for c in (0, 2, 1, 3):
            ph2[c].wait_recv()
            out_ref[0, :, c * QC:(c + 1) * QC] = (
                acc1[c] + rbuf[4 + c].astype(jnp.float32))

        for r in ph1 + ph2:
            r.wait_send()

    return pl.pallas_call(
        body,
        out_shape=jax.ShapeDtypeStruct((1, SQ, DM), jnp.float32),
        in_specs=[
            pl.BlockSpec(memory_space=pltpu.MemorySpace.VMEM),
            pl.BlockSpec(memory_space=pltpu.MemorySpace.VMEM),
            pl.BlockSpec(memory_space=pltpu.MemorySpace.HBM),
            pl.BlockSpec(memory_space=pltpu.MemorySpace.HBM),
            pl.BlockSpec(memory_space=pltpu.MemorySpace.VMEM),
        ],
        out_specs=pl.BlockSpec(memory_space=pltpu.MemorySpace.VMEM),
        scratch_shapes=[
            pltpu.VMEM((HQ_PER, SKV, DH), jnp.float32),
            pltpu.VMEM((HQ_PER, SKV, DH), jnp.float32),
            pltpu.VMEM((8, SQ, DM // 4), jnp.bfloat16),
            pltpu.VMEM((8, SQ, DM // 4), jnp.bfloat16),
            pltpu.SemaphoreType.DMA((2, HQ_PER)),
            pltpu.SemaphoreType.DMA((8,)),
            pltpu.SemaphoreType.DMA((8,)),
        ],
        compiler_params=pltpu.CompilerParams(
            collective_id=0, vmem_limit_bytes=100 * 1024 * 1024),
    )(x, Wq, K_ext, V_ext, Wo)
